# Initial kernel scaffold; baseline (speedup 1.0000x reference)
import jax
import jax.numpy as jnp
from jax.experimental import pallas as pl


def kernel(hidden_states, W_out, b_out, W_in, b_in, Wg, Wu, Wd):
    raise NotImplementedError("write your pallas kernel here")


# speedup vs baseline: 5.1039x; 5.1039x over previous
import jax
import jax.numpy as jnp
from jax.experimental import pallas as pl
from jax.experimental.pallas import tpu as pltpu

_G = 8
_INNER = 8
_TOPG = 2
_TOPI = 4


def _router_body(x_ref, wout_ref, bout_ref, win_ref, bin_ref,
                 selw_ref, wfull_ref):
    x = x_ref[...]
    lo = jnp.dot(x, wout_ref[...],
                 preferred_element_type=jnp.float32) + bout_ref[...]
    iota_g = jax.lax.broadcasted_iota(jnp.int32, lo.shape, 1)
    neg = jnp.float32(-jnp.inf)

    m1 = jnp.max(lo, axis=1, keepdims=True)
    i1 = jnp.min(jnp.where(lo == m1, iota_g, 127), axis=1, keepdims=True)
    lom = jnp.where(iota_g == i1, neg, lo)
    m2 = jnp.max(lom, axis=1, keepdims=True)
    i2 = jnp.min(jnp.where(lom == m2, iota_g, 127), axis=1, keepdims=True)
    w1 = jax.nn.sigmoid(m1 - m2)
    w2 = 1.0 - w1
    iota_s = jax.lax.broadcasted_iota(jnp.int32, selw_ref.shape, 1)
    selw_ref[...] = jnp.where(iota_s < _TOPI, w1, w2)

    n_tile = x.shape[0]
    acc = jnp.zeros((n_tile, _G * _INNER), jnp.float32)
    iota_e = jax.lax.broadcasted_iota(jnp.int32, acc.shape, 1)
    for g in range(_G):
        il = jnp.dot(x, win_ref[g],
                     preferred_element_type=jnp.float32) + bin_ref[g:g + 1, :]
        work = il
        for _ in range(_TOPI):
            mk = jnp.max(work, axis=1, keepdims=True)
            ik = jnp.min(jnp.where(work == mk, iota_g, 127), axis=1,
                         keepdims=True)
            work = jnp.where(iota_g == ik, neg, work)
            eid = g * _INNER + ik
            hit = iota_e == eid
            acc = (acc
                   + jnp.where(hit & (i1 == g), w1, 0.0)
                   + jnp.where(hit & (i2 == g), w2, 0.0))
    wfull_ref[...] = acc


def _expert_body(xb_ref, wfull_ref, wg_ref, wu_ref, wd_ref, out_ref):
    e = pl.program_id(1)
    wf = wfull_ref[...]
    iota_e = jax.lax.broadcasted_iota(jnp.int32, wf.shape, 1)
    wcol = jnp.sum(jnp.where(iota_e == e, wf, 0.0), axis=1, keepdims=True)
    xb = xb_ref[...]
    g = jnp.dot(xb, wg_ref[0], preferred_element_type=jnp.float32)
    u = jnp.dot(xb, wu_ref[0], preferred_element_type=jnp.float32)
    h = (g * jax.nn.sigmoid(g) * u).astype(jnp.bfloat16)
    d = jnp.dot(h, wd_ref[0], preferred_element_type=jnp.float32)

    @pl.when(e == 0)
    def _():
        out_ref[...] = jnp.zeros_like(out_ref)

    out_ref[...] += d * wcol


def kernel(hidden_states, W_out, b_out, W_in, b_in, Wg, Wu, Wd):
    Bx, Sx, Dx = hidden_states.shape
    N = Bx * Sx
    E, _, DFF = Wg.shape
    x = hidden_states.reshape(N, Dx)

    TN = 512
    selw, wfull = pl.pallas_call(
        _router_body,
        grid=(N // TN,),
        in_specs=[
            pl.BlockSpec((TN, Dx), lambda n: (n, 0)),
            pl.BlockSpec((Dx, _G), lambda n: (0, 0)),
            pl.BlockSpec((1, _G), lambda n: (0, 0)),
            pl.BlockSpec((_G, Dx, _INNER), lambda n: (0, 0, 0)),
            pl.BlockSpec((_G, _INNER), lambda n: (0, 0)),
        ],
        out_specs=[
            pl.BlockSpec((TN, _TOPG * _TOPI), lambda n: (n, 0)),
            pl.BlockSpec((TN, E), lambda n: (n, 0)),
        ],
        out_shape=[
            jax.ShapeDtypeStruct((N, _TOPG * _TOPI), jnp.float32),
            jax.ShapeDtypeStruct((N, E), jnp.float32),
        ],
    )(x, W_out, b_out.reshape(1, _G), W_in, b_in)

    xb = x.astype(jnp.bfloat16)
    Wgb = Wg.astype(jnp.bfloat16)
    Wub = Wu.astype(jnp.bfloat16)
    Wdb = Wd.astype(jnp.bfloat16)

    TNE = 1024
    final = pl.pallas_call(
        _expert_body,
        grid=(N // TNE, E),
        in_specs=[
            pl.BlockSpec((TNE, Dx), lambda n, e: (n, 0)),
            pl.BlockSpec((TNE, E), lambda n, e: (n, 0)),
            pl.BlockSpec((1, Dx, DFF), lambda n, e: (e, 0, 0)),
            pl.BlockSpec((1, Dx, DFF), lambda n, e: (e, 0, 0)),
            pl.BlockSpec((1, DFF, Dx), lambda n, e: (e, 0, 0)),
        ],
        out_specs=pl.BlockSpec((TNE, Dx), lambda n, e: (n, 0)),
        out_shape=jax.ShapeDtypeStruct((N, Dx), jnp.float32),
        compiler_params=pltpu.CompilerParams(
            dimension_semantics=("parallel", "arbitrary")),
    )(xb, wfull, Wgb, Wub, Wdb)

    return final.reshape(Bx, Sx, Dx), selw
